# Initial kernel scaffold; baseline (speedup 1.0000x reference)
import jax
import jax.numpy as jnp
from jax.experimental import pallas as pl


def kernel(x, edge_index_rsr, edge_index_rtr, edge_index_rur, W1_rsr, b1_rsr, W1_rtr, b1_rtr, W1_rur, b1_rur, W2_rsr, b2_rsr, W2_rtr, b2_rtr, W2_rur, b2_rur):
    raise NotImplementedError("write your pallas kernel here")


# speedup vs baseline: 11.0879x; 11.0879x over previous
import functools

import jax
import jax.numpy as jnp
from jax import lax
from jax.experimental import pallas as pl
from jax.experimental.pallas import tpu as pltpu
from jax.experimental.pallas import tpu_sc as plsc

N = 10000
EDGES = 320000
HID = 16
NCORE = 2
NSUB = 16
NTILE = NCORE * NSUB
BLK = 128
BPT = 79
EPT = BPT * BLK
EPAD = NTILE * EPT
NPAD = 10112
RPT = NPAD // NSUB

_f32 = jnp.float32


def _mesh():
    return plsc.VectorSubcoreMesh(core_axis_name="c", subcore_axis_name="s")


_SC_PARAMS = pltpu.CompilerParams(use_tc_tiling_on_sc=False)


def _fill_rows(ref, nrows, val):
    @pl.loop(0, nrows)
    def _(r):
        ref.at[r][...] = jnp.full((16,), val, _f32)


def _deg_sc_body(idx_hbm, out_hbm, idx_v, hot_v, zbuf, acc):
    cid = lax.axis_index("c")
    sid = lax.axis_index("s")
    wid = sid * NCORE + cid
    _fill_rows(zbuf, RPT, 0.0)
    pltpu.sync_copy(zbuf, acc.at[pl.ds(sid * RPT, RPT)])
    plsc.subcore_barrier()
    for a in range(6):
        onehot = jnp.where(lax.iota(jnp.int32, 16) == a, 1.0, 0.0).astype(_f32)

        @pl.loop(0, BLK)
        def _(r, v=onehot):
            hot_v.at[r][...] = v

        pltpu.sync_copy(idx_hbm.at[a, wid], idx_v)

        @pl.loop(0, BPT)
        def _(j):
            pltpu.sync_copy(hot_v, acc.at[idx_v.at[j]], add=True)

    plsc.subcore_barrier()
    pltpu.sync_copy(acc.at[pl.ds(sid * RPT, RPT)],
                    out_hbm.at[cid, pl.ds(sid * RPT, RPT)])


def _deg_sc(idx_all):
    return pl.kernel(
        _deg_sc_body,
        mesh=_mesh(),
        out_type=jax.ShapeDtypeStruct((NCORE, NPAD, HID), _f32),
        scratch_types=[pltpu.VMEM((BPT, BLK), jnp.int32),
                       pltpu.VMEM((BLK, HID), _f32),
                       pltpu.VMEM((RPT, HID), _f32),
                       pltpu.VMEM_SHARED((NPAD, HID), _f32)],
        compiler_params=_SC_PARAMS,
    )(idx_all)


def _agg_sc_body(t0, t1, t2, idx_hbm, out_hbm, src_v, dst_v, rows_v, zbuf,
                 a0, a1, a2):
    tabs = [t0, t1, t2]
    accs = [a0, a1, a2]
    cid = lax.axis_index("c")
    sid = lax.axis_index("s")
    wid = sid * NCORE + cid
    _fill_rows(zbuf, RPT, 0.0)
    for e in range(3):
        pltpu.sync_copy(zbuf, accs[e].at[pl.ds(sid * RPT, RPT)])
    plsc.subcore_barrier()
    for e in range(3):
        pltpu.sync_copy(idx_hbm.at[e, wid], src_v)
        pltpu.sync_copy(idx_hbm.at[3 + e, wid], dst_v)

        @pl.loop(0, BPT)
        def _(j, tab=tabs[e], acc=accs[e]):
            pltpu.sync_copy(tab.at[src_v.at[j]], rows_v)
            pltpu.sync_copy(rows_v, acc.at[dst_v.at[j]], add=True)

    plsc.subcore_barrier()
    for e in range(3):
        pltpu.sync_copy(accs[e].at[pl.ds(sid * RPT, RPT)],
                        out_hbm.at[cid, e, pl.ds(sid * RPT, RPT)])


def _agg_sc(t0, t1, t2, idx_all):
    return pl.kernel(
        _agg_sc_body,
        mesh=_mesh(),
        out_type=jax.ShapeDtypeStruct((NCORE, 3, NPAD, HID), _f32),
        scratch_types=[pltpu.VMEM((BPT, BLK), jnp.int32),
                       pltpu.VMEM((BPT, BLK), jnp.int32),
                       pltpu.VMEM((BLK, HID), _f32),
                       pltpu.VMEM((RPT, HID), _f32)]
        + [pltpu.VMEM_SHARED((NPAD, HID), _f32)] * 3,
        compiler_params=_SC_PARAMS,
    )(t0, t1, t2, idx_all)


def _mm_body(x_ref, w_ref, o_ref):
    o_ref[...] = jnp.dot(x_ref[...], w_ref[...],
                         preferred_element_type=_f32,
                         precision=lax.Precision.HIGHEST)


def _h1_tc(x, w):
    return pl.pallas_call(
        _mm_body,
        grid=(10,),
        in_specs=[pl.BlockSpec((1000, 128), lambda i: (i, 0)),
                  pl.BlockSpec((128, 48), lambda i: (0, 0))],
        out_specs=pl.BlockSpec((1000, 48), lambda i: (i, 0)),
        out_shape=jax.ShapeDtypeStruct((N, 48), _f32),
    )(x, w)


def _inv_sqrt_deg(d, col):
    deg = d[:, col:col + 1]
    return lax.rsqrt(jnp.maximum(deg, 1.0))


def _scale_body(h_ref, d_ref, o_ref):
    h = h_ref[...]
    d = d_ref[0] + d_ref[1]
    for e in range(3):
        o_ref[e] = h[:, 16 * e:16 * (e + 1)] * _inv_sqrt_deg(d, e)


def _scale_tc(h1, degs):
    return pl.pallas_call(
        _scale_body,
        grid=(10,),
        in_specs=[pl.BlockSpec((1000, 48), lambda i: (i, 0)),
                  pl.BlockSpec((NCORE, 1000, HID), lambda i: (0, i, 0))],
        out_specs=pl.BlockSpec((3, 1000, HID), lambda i: (0, i, 0)),
        out_shape=jax.ShapeDtypeStruct((3, N, HID), _f32),
    )(h1, degs)


def _mid_body(a_ref, d_ref, b_ref, o_ref):
    d = d_ref[0] + d_ref[1]
    h = jnp.broadcast_to(b_ref[0:1, :], (1000, HID))
    for e in range(3):
        h = h + (a_ref[0, e] + a_ref[1, e]) * _inv_sqrt_deg(d, 3 + e)
    h = jnp.maximum(h, 0.0)
    for e in range(3):
        o_ref[e] = h * _inv_sqrt_deg(d, e)


def _mid_tc(aggs, degs, bsum):
    return pl.pallas_call(
        _mid_body,
        grid=(10,),
        in_specs=[pl.BlockSpec((NCORE, 3, 1000, HID), lambda i: (0, 0, i, 0)),
                  pl.BlockSpec((NCORE, 1000, HID), lambda i: (0, i, 0)),
                  pl.BlockSpec((8, HID), lambda i: (0, 0))],
        out_specs=pl.BlockSpec((3, 1000, HID), lambda i: (0, i, 0)),
        out_shape=jax.ShapeDtypeStruct((3, N, HID), _f32),
    )(aggs, degs, bsum)


def _out_body(a_ref, d_ref, w_ref, b_ref, o_ref):
    d = d_ref[0] + d_ref[1]
    ms = []
    for e in range(3):
        ms.append((a_ref[0, e] + a_ref[1, e]) * _inv_sqrt_deg(d, 3 + e))
    m = jnp.concatenate(ms, axis=1)
    o_ref[...] = (jnp.dot(m, w_ref[...], preferred_element_type=_f32,
                          precision=lax.Precision.HIGHEST)
                  + b_ref[0:1, :])


def _out_tc(aggs, degs, w2, b2):
    return pl.pallas_call(
        _out_body,
        grid=(10,),
        in_specs=[pl.BlockSpec((NCORE, 3, 1000, HID), lambda i: (0, 0, i, 0)),
                  pl.BlockSpec((NCORE, 1000, HID), lambda i: (0, i, 0)),
                  pl.BlockSpec((48, 2), lambda i: (0, 0)),
                  pl.BlockSpec((8, 2), lambda i: (0, 0))],
        out_specs=pl.BlockSpec((1000, 2), lambda i: (i, 0)),
        out_shape=jax.ShapeDtypeStruct((N, 2), _f32),
    )(aggs, degs, w2, b2)


def kernel(x, edge_index_rsr, edge_index_rtr, edge_index_rur,
           W1_rsr, b1_rsr, W1_rtr, b1_rtr, W1_rur, b1_rur,
           W2_rsr, b2_rsr, W2_rtr, b2_rtr, W2_rur, b2_rur):
    eis = [edge_index_rsr, edge_index_rtr, edge_index_rur]
    pads = jnp.full((EPAD - EDGES,), N, dtype=jnp.int32)
    idx_all = jnp.stack(
        [jnp.concatenate([ei[r], pads]).reshape(NTILE, BPT, BLK)
         for r in range(2) for ei in eis])

    w1 = jnp.concatenate([W1_rsr, W1_rtr, W1_rur], axis=1)
    degs = _deg_sc(idx_all)
    h1 = _h1_tc(x, w1)
    t1 = _scale_tc(h1, degs)
    t1 = jnp.pad(t1, ((0, 0), (0, NPAD - N), (0, 0)))
    a1 = _agg_sc(t1[0], t1[1], t1[2], idx_all)

    bsum1 = jnp.broadcast_to(b1_rsr + b1_rtr + b1_rur, (8, HID))
    t2 = _mid_tc(a1, degs, bsum1)
    t2 = jnp.pad(t2, ((0, 0), (0, NPAD - N), (0, 0)))
    a2 = _agg_sc(t2[0], t2[1], t2[2], idx_all)

    w2 = jnp.concatenate([W2_rsr, W2_rtr, W2_rur], axis=0)
    bsum2 = jnp.broadcast_to(b2_rsr + b2_rtr + b2_rur, (8, 2))
    return _out_tc(a2, degs, w2, bsum2)


# speedup vs baseline: 13.8431x; 1.2485x over previous
import functools

import jax
import jax.numpy as jnp
from jax import lax
from jax.experimental import pallas as pl
from jax.experimental.pallas import tpu as pltpu
from jax.experimental.pallas import tpu_sc as plsc

N = 10000
EDGES = 320000
HID = 16
NCORE = 2
NSUB = 16
NTILE = NCORE * NSUB
BLK = 128
BPT = 80
KBUF = 8
NBATCH = BPT // KBUF
EPT = BPT * BLK
EPAD = NTILE * EPT
NPAD = 10112
RPT = NPAD // NSUB

_f32 = jnp.float32


def _mesh():
    return plsc.VectorSubcoreMesh(core_axis_name="c", subcore_axis_name="s")


_SC_PARAMS = pltpu.CompilerParams(use_tc_tiling_on_sc=False)


def _fill_rows(ref, nrows, val):
    @pl.loop(0, nrows)
    def _(r):
        ref.at[r][...] = jnp.full((16,), val, _f32)


def _deg_sc_body(idx_hbm, out_hbm, i0, i1, i2, i3, i4, i5,
                 h0, h1, h2, h3, h4, h5, zbuf, acc, ssem):
    idxs = [i0, i1, i2, i3, i4, i5]
    hots = [h0, h1, h2, h3, h4, h5]
    cid = lax.axis_index("c")
    sid = lax.axis_index("s")
    wid = sid * NCORE + cid
    _fill_rows(zbuf, RPT, 0.0)
    pltpu.sync_copy(zbuf, acc.at[pl.ds(sid * RPT, RPT)])
    for a in range(6):
        onehot = jnp.where(lax.iota(jnp.int32, 16) == a, 1.0, 0.0).astype(_f32)

        @pl.loop(0, BLK)
        def _(r, v=onehot, hot=hots[a]):
            hot.at[r][...] = v

        pltpu.sync_copy(idx_hbm.at[a, wid], idxs[a])
    plsc.subcore_barrier()
    for a in range(6):

        @pl.loop(0, BPT)
        def _(j, hot=hots[a], idx=idxs[a]):
            pltpu.async_copy(hot, acc.at[idx.at[j]], ssem, add=True)

    @pl.loop(0, 6 * BPT)
    def _(j):
        pltpu.make_async_copy(out_hbm.at[cid, pl.ds(0, BLK)], h0, ssem).wait()

    plsc.subcore_barrier()
    pltpu.sync_copy(acc.at[pl.ds(sid * RPT, RPT)],
                    out_hbm.at[cid, pl.ds(sid * RPT, RPT)])


def _deg_sc(idx_all):
    return pl.kernel(
        _deg_sc_body,
        mesh=_mesh(),
        out_type=jax.ShapeDtypeStruct((NCORE, NPAD, HID), _f32),
        scratch_types=[pltpu.VMEM((BPT, BLK), jnp.int32)] * 6
        + [pltpu.VMEM((BLK, HID), _f32)] * 6
        + [pltpu.VMEM((RPT, HID), _f32),
           pltpu.VMEM_SHARED((NPAD, HID), _f32),
           pltpu.SemaphoreType.DMA],
        compiler_params=_SC_PARAMS,
    )(idx_all)


def _agg_sc_body(t0, t1, t2, idx_hbm, out_hbm, src_v, dst_v,
                 r0, r1, r2, r3, r4, r5, r6, r7, zbuf,
                 a0, a1, a2, gsem, ssem):
    tabs = [t0, t1, t2]
    accs = [a0, a1, a2]
    rows = [r0, r1, r2, r3, r4, r5, r6, r7]
    cid = lax.axis_index("c")
    sid = lax.axis_index("s")
    wid = sid * NCORE + cid
    _fill_rows(zbuf, RPT, 0.0)
    for e in range(3):
        pltpu.sync_copy(zbuf, accs[e].at[pl.ds(sid * RPT, RPT)])
    plsc.subcore_barrier()
    for e in range(3):
        pltpu.sync_copy(idx_hbm.at[e, wid], src_v)
        pltpu.sync_copy(idx_hbm.at[3 + e, wid], dst_v)

        @pl.loop(0, NBATCH)
        def _(b, tab=tabs[e], acc=accs[e]):
            base = b * KBUF
            gets = [pltpu.async_copy(tab.at[src_v.at[base + i]], rows[i], gsem)
                    for i in range(KBUF)]
            puts = []
            for i in range(KBUF):
                gets[i].wait()
                puts.append(pltpu.async_copy(rows[i], acc.at[dst_v.at[base + i]],
                                             ssem, add=True))
            for p in puts:
                p.wait()

    plsc.subcore_barrier()
    for e in range(3):
        pltpu.sync_copy(accs[e].at[pl.ds(sid * RPT, RPT)],
                        out_hbm.at[cid, e, pl.ds(sid * RPT, RPT)])


def _agg_sc(t0, t1, t2, idx_all):
    return pl.kernel(
        _agg_sc_body,
        mesh=_mesh(),
        out_type=jax.ShapeDtypeStruct((NCORE, 3, NPAD, HID), _f32),
        scratch_types=[pltpu.VMEM((BPT, BLK), jnp.int32),
                       pltpu.VMEM((BPT, BLK), jnp.int32)]
        + [pltpu.VMEM((BLK, HID), _f32)] * KBUF
        + [pltpu.VMEM((RPT, HID), _f32)]
        + [pltpu.VMEM_SHARED((NPAD, HID), _f32)] * 3
        + [pltpu.SemaphoreType.DMA, pltpu.SemaphoreType.DMA],
        compiler_params=_SC_PARAMS,
    )(t0, t1, t2, idx_all)


def _mm_body(x_ref, w_ref, o_ref):
    o_ref[...] = jnp.dot(x_ref[...], w_ref[...],
                         preferred_element_type=_f32,
                         precision=lax.Precision.HIGHEST)


def _h1_tc(x, w):
    return pl.pallas_call(
        _mm_body,
        grid=(10,),
        in_specs=[pl.BlockSpec((1000, 128), lambda i: (i, 0)),
                  pl.BlockSpec((128, 48), lambda i: (0, 0))],
        out_specs=pl.BlockSpec((1000, 48), lambda i: (i, 0)),
        out_shape=jax.ShapeDtypeStruct((N, 48), _f32),
    )(x, w)


def _inv_sqrt_deg(d, col):
    deg = d[:, col:col + 1]
    return lax.rsqrt(jnp.maximum(deg, 1.0))


def _scale_body(h_ref, d_ref, o_ref):
    h = h_ref[...]
    d = d_ref[0] + d_ref[1]
    for e in range(3):
        o_ref[e] = h[:, 16 * e:16 * (e + 1)] * _inv_sqrt_deg(d, e)


def _scale_tc(h1, degs):
    return pl.pallas_call(
        _scale_body,
        grid=(10,),
        in_specs=[pl.BlockSpec((1000, 48), lambda i: (i, 0)),
                  pl.BlockSpec((NCORE, 1000, HID), lambda i: (0, i, 0))],
        out_specs=pl.BlockSpec((3, 1000, HID), lambda i: (0, i, 0)),
        out_shape=jax.ShapeDtypeStruct((3, N, HID), _f32),
    )(h1, degs)


def _mid_body(a_ref, d_ref, b_ref, o_ref):
    d = d_ref[0] + d_ref[1]
    h = jnp.broadcast_to(b_ref[0:1, :], (1000, HID))
    for e in range(3):
        h = h + (a_ref[0, e] + a_ref[1, e]) * _inv_sqrt_deg(d, 3 + e)
    h = jnp.maximum(h, 0.0)
    for e in range(3):
        o_ref[e] = h * _inv_sqrt_deg(d, e)


def _mid_tc(aggs, degs, bsum):
    return pl.pallas_call(
        _mid_body,
        grid=(10,),
        in_specs=[pl.BlockSpec((NCORE, 3, 1000, HID), lambda i: (0, 0, i, 0)),
                  pl.BlockSpec((NCORE, 1000, HID), lambda i: (0, i, 0)),
                  pl.BlockSpec((8, HID), lambda i: (0, 0))],
        out_specs=pl.BlockSpec((3, 1000, HID), lambda i: (0, i, 0)),
        out_shape=jax.ShapeDtypeStruct((3, N, HID), _f32),
    )(aggs, degs, bsum)


def _out_body(a_ref, d_ref, w_ref, b_ref, o_ref):
    d = d_ref[0] + d_ref[1]
    ms = []
    for e in range(3):
        ms.append((a_ref[0, e] + a_ref[1, e]) * _inv_sqrt_deg(d, 3 + e))
    m = jnp.concatenate(ms, axis=1)
    o_ref[...] = (jnp.dot(m, w_ref[...], preferred_element_type=_f32,
                          precision=lax.Precision.HIGHEST)
                  + b_ref[0:1, :])


def _out_tc(aggs, degs, w2, b2):
    return pl.pallas_call(
        _out_body,
        grid=(10,),
        in_specs=[pl.BlockSpec((NCORE, 3, 1000, HID), lambda i: (0, 0, i, 0)),
                  pl.BlockSpec((NCORE, 1000, HID), lambda i: (0, i, 0)),
                  pl.BlockSpec((48, 2), lambda i: (0, 0)),
                  pl.BlockSpec((8, 2), lambda i: (0, 0))],
        out_specs=pl.BlockSpec((1000, 2), lambda i: (i, 0)),
        out_shape=jax.ShapeDtypeStruct((N, 2), _f32),
    )(aggs, degs, w2, b2)


def kernel(x, edge_index_rsr, edge_index_rtr, edge_index_rur,
           W1_rsr, b1_rsr, W1_rtr, b1_rtr, W1_rur, b1_rur,
           W2_rsr, b2_rsr, W2_rtr, b2_rtr, W2_rur, b2_rur):
    eis = [edge_index_rsr, edge_index_rtr, edge_index_rur]
    pads = jnp.full((EPAD - EDGES,), N, dtype=jnp.int32)
    idx_all = jnp.stack(
        [jnp.concatenate([ei[r], pads]).reshape(NTILE, BPT, BLK)
         for r in range(2) for ei in eis])

    w1 = jnp.concatenate([W1_rsr, W1_rtr, W1_rur], axis=1)
    degs = _deg_sc(idx_all)
    h1 = _h1_tc(x, w1)
    t1 = _scale_tc(h1, degs)
    t1 = jnp.pad(t1, ((0, 0), (0, NPAD - N), (0, 0)))
    a1 = _agg_sc(t1[0], t1[1], t1[2], idx_all)

    bsum1 = jnp.broadcast_to(b1_rsr + b1_rtr + b1_rur, (8, HID))
    t2 = _mid_tc(a1, degs, bsum1)
    t2 = jnp.pad(t2, ((0, 0), (0, NPAD - N), (0, 0)))
    a2 = _agg_sc(t2[0], t2[1], t2[2], idx_all)

    w2 = jnp.concatenate([W2_rsr, W2_rtr, W2_rur], axis=0)
    bsum2 = jnp.broadcast_to(b2_rsr + b2_rtr + b2_rur, (8, 2))
    return _out_tc(a2, degs, w2, bsum2)
